# baseline (speedup 1.0000x reference)
import functools

import jax
import jax.numpy as jnp
from jax import lax
from jax.experimental import pallas as pl
from jax.experimental.pallas import tpu as pltpu
from jax.experimental.pallas import tpu_sc as plsc

N = 10000
C = 128
T = 12
E = 320000

NUM_SC = 2
NUM_TILES = 16
B = 128
SLAB = 4
NSL = 40
NB = NSL * SLAB
PER_TILE = NB * B
EP = NUM_TILES * PER_TILE
NP = 10112
ROWS_W = NP // NUM_TILES
CH = C // NUM_SC
DEG_W = 16
NBUF = 4
TCH = 6

BN = 1264
NBLK = NP // BN


def _tc_a_body(xt_ref, xp_ref, d0_ref, d1_ref, wf1_ref, wf0_ref, wg1_ref,
               wg0_ref, bf_ref, bg_ref, gw_ref, out_ref, *, first):
  t = pl.program_id(0)
  xt = xt_ref[0]
  if first:
    fac = jnp.where(t == 0, 0.0, 1.0).astype(jnp.float32)
    xp = xp_ref[0] * fac
  else:
    xp = xp_ref[0]
  dot = functools.partial(jnp.dot, preferred_element_type=jnp.float32)
  fpre = dot(xt, wf1_ref[...]) + dot(xp, wf0_ref[...]) + bf_ref[...]
  gpre = dot(xt, wg1_ref[...]) + dot(xp, wg0_ref[...]) + bg_ref[...]
  h = jnp.tanh(fpre) * jax.nn.sigmoid(gpre)
  xw = dot(h, gw_ref[...])
  dinv = lax.rsqrt(d0_ref[...] + d1_ref[...] + 1.0)
  xws = (xw * dinv).astype(jnp.bfloat16)
  out_ref[0, 0] = xws[:, :CH]
  out_ref[1, 0] = xws[:, CH:]


def _tc_a(xT, d0, d1, wf1, wf0, wg1, wg0, bf, bg, gw, toff, first):
  dblk = pl.BlockSpec((BN, 1), lambda t, nb: (nb, 0))
  wblk = pl.BlockSpec((C, C), lambda t, nb: (0, 0))
  bblk = pl.BlockSpec((1, C), lambda t, nb: (0, 0))
  if first:
    pmap = lambda t, nb: (jnp.where(t == 0, 0, t - 1), nb, 0)
  else:
    pmap = lambda t, nb: (t + toff - 1, nb, 0)
  return pl.pallas_call(
      functools.partial(_tc_a_body, first=first),
      grid=(TCH, NBLK),
      in_specs=[
          pl.BlockSpec((1, BN, C), lambda t, nb: (t + toff, nb, 0)),
          pl.BlockSpec((1, BN, C), pmap),
          dblk, dblk, wblk, wblk, wblk, wblk, bblk, bblk, wblk,
      ],
      out_specs=pl.BlockSpec((NUM_SC, 1, BN, CH), lambda t, nb: (0, t, nb, 0)),
      out_shape=jax.ShapeDtypeStruct((NUM_SC, TCH, NP, CH), jnp.bfloat16),
  )(xT, xT, d0, d1, wf1, wf0, wg1, wg0, bf, bg, gw)


def _sc_deg_body(dst_hbm, out_hbm, dstv, ones_v, zero_v, agg_sp, sem):
  cid = lax.axis_index("c")
  sid = lax.axis_index("s")
  pltpu.sync_copy(dst_hbm.at[sid], dstv)

  def fill_ones(i, _):
    ones_v[i, :] = jnp.full((DEG_W,), 1.0, jnp.float32)
    return 0
  lax.fori_loop(0, B, fill_ones, 0)

  def fill_zero(i, _):
    zero_v[i, :] = jnp.zeros((DEG_W,), jnp.float32)
    return 0
  lax.fori_loop(0, ROWS_W, fill_zero, 0)
  pltpu.sync_copy(zero_v, agg_sp.at[pl.ds(sid * ROWS_W, ROWS_W)])
  plsc.subcore_barrier()

  nb0 = NB // 2
  lo = jnp.where(cid == 0, 0, nb0)
  hi = jnp.where(cid == 0, nb0, NB)

  def batch(j, _):
    pltpu.sync_copy(ones_v, agg_sp.at[dstv.at[j]], add=True)
    return 0
  lax.fori_loop(lo, hi, batch, 0)
  plsc.subcore_barrier()
  pltpu.sync_copy(agg_sp.at[pl.ds(sid * ROWS_W, ROWS_W)],
                  out_hbm.at[cid].at[pl.ds(sid * ROWS_W, ROWS_W)])


def _sc_deg(dst_pad):
  mesh = plsc.VectorSubcoreMesh(core_axis_name="c", subcore_axis_name="s")
  return pl.kernel(
      _sc_deg_body,
      compiler_params=pltpu.CompilerParams(use_tc_tiling_on_sc=False),
      out_type=jax.ShapeDtypeStruct((NUM_SC, NP, DEG_W), jnp.float32),
      mesh=mesh,
      scratch_types=[
          pltpu.VMEM((NB, B), jnp.int32),
          pltpu.VMEM((B, DEG_W), jnp.float32),
          pltpu.VMEM((ROWS_W, DEG_W), jnp.float32),
          pltpu.VMEM_SHARED((NP, DEG_W), jnp.float32),
          pltpu.SemaphoreType.DMA,
      ],
  )(dst_pad)


def _sc_main_body(xws_hbm, src_hbm, dst_hbm, agg_hbm, srcf, dstv, rows, agg_sp,
                  gsem, ssem):
  cid = lax.axis_index("c")
  sid = lax.axis_index("s")
  pltpu.sync_copy(src_hbm.at[sid], srcf)
  pltpu.sync_copy(dst_hbm.at[sid], dstv)

  def per_t(t, _):
    pltpu.sync_copy(xws_hbm.at[cid].at[t].at[pl.ds(sid * ROWS_W, ROWS_W)],
                    agg_sp.at[pl.ds(sid * ROWS_W, ROWS_W)])
    plsc.subcore_barrier()

    tbl = xws_hbm.at[cid].at[t]

    def slab_gather(jj, p):
      pltpu.async_copy(tbl.at[srcf.at[pl.ds(jj * SLAB * B, SLAB * B)]],
                       rows.at[p], gsem.at[p])

    def slab_scatters(jj, p, fire):
      for k in range(SLAB):
        sub = rows.at[p].at[pl.ds(k * B, B)]
        dsti = agg_sp.at[dstv.at[jj * SLAB + k]]
        if fire:
          pltpu.async_copy(sub, dsti, ssem.at[p], add=True)
        else:
          pltpu.make_async_copy(sub, dsti, ssem.at[p]).wait()

    for jj in range(NBUF - 1):
      slab_gather(jj, jj)

    def batch(jj, _):
      p = jj % NBUF
      pr = (jj + NBUF - 1) % NBUF

      @pl.when(jj > 0)
      def _():
        slab_scatters(jj - 1, pr, False)

      @pl.when(jj + NBUF - 1 < NSL)
      def _():
        slab_gather(jj + NBUF - 1, pr)

      pltpu.make_async_copy(tbl.at[srcf.at[pl.ds(jj * SLAB * B, SLAB * B)]],
                            rows.at[p], gsem.at[p]).wait()
      slab_scatters(jj, p, True)
      return 0
    lax.fori_loop(0, NSL, batch, 0)
    slab_scatters(NSL - 1, (NSL - 1) % NBUF, False)
    plsc.subcore_barrier()
    pltpu.sync_copy(agg_sp.at[pl.ds(sid * ROWS_W, ROWS_W)],
                    agg_hbm.at[cid].at[t].at[pl.ds(sid * ROWS_W, ROWS_W)])
    return 0

  lax.fori_loop(0, TCH, per_t, 0)


def _sc_main(xws, src_pad, dst_pad):
  mesh = plsc.VectorSubcoreMesh(core_axis_name="c", subcore_axis_name="s")
  return pl.kernel(
      _sc_main_body,
      compiler_params=pltpu.CompilerParams(use_tc_tiling_on_sc=False),
      out_type=jax.ShapeDtypeStruct((NUM_SC, TCH, NP, CH), jnp.bfloat16),
      mesh=mesh,
      scratch_types=[
          pltpu.VMEM((PER_TILE,), jnp.int32),
          pltpu.VMEM((NB, B), jnp.int32),
          pltpu.VMEM((NBUF, SLAB * B, CH), jnp.bfloat16),
          pltpu.VMEM_SHARED((NP, CH), jnp.bfloat16),
          pltpu.SemaphoreType.DMA((NBUF,)),
          pltpu.SemaphoreType.DMA((NBUF,)),
      ],
  )(xws, src_pad.reshape(NUM_TILES, PER_TILE), dst_pad)


def _tc_c_body(agg_ref, xt_ref, d0_ref, d1_ref, b_ref, out_ref):
  dinv = lax.rsqrt(d0_ref[...] + d1_ref[...] + 1.0)
  agg = jnp.concatenate([agg_ref[0, 0], agg_ref[1, 0]], axis=-1).astype(jnp.float32)
  out_ref[0] = agg * dinv + b_ref[...] + xt_ref[0]


def _tc_c(agg, xT, d0, d1, b, toff):
  dblk = pl.BlockSpec((BN, 1), lambda t, nb: (nb, 0))
  bblk = pl.BlockSpec((1, C), lambda t, nb: (0, 0))
  return pl.pallas_call(
      _tc_c_body,
      grid=(TCH, NBLK),
      in_specs=[pl.BlockSpec((NUM_SC, 1, BN, CH), lambda t, nb: (0, t, nb, 0)),
                pl.BlockSpec((1, BN, C), lambda t, nb: (t + toff, nb, 0)),
                dblk, dblk, bblk],
      out_specs=pl.BlockSpec((1, BN, C), lambda t, nb: (t, nb, 0)),
      out_shape=jax.ShapeDtypeStruct((TCH, NP, C), jnp.float32),
  )(agg, xT, d0, d1, b)


@jax.jit
def kernel(x, edge_index, filter_W, filter_b, gate_W, gate_b, gcn_W, gcn_b):
  xT = jnp.transpose(x, (2, 0, 1))
  xTp = jnp.pad(xT, ((0, 0), (0, NP - N), (0, 0)))

  src = edge_index[0]
  dst = edge_index[1]
  pad = EP - E
  src_pad = jnp.concatenate([src, jnp.zeros((pad,), jnp.int32)])
  dst_pad = jnp.concatenate([dst, jnp.full((pad,), N, jnp.int32)])
  src_pad = src_pad.reshape(NUM_TILES, NB, B)
  dst_pad = dst_pad.reshape(NUM_TILES, NB, B)

  deg_parts = _sc_deg(dst_pad)
  d0 = deg_parts[0, :, 0:1]
  d1 = deg_parts[1, :, 0:1]

  wf1 = filter_W[:, :, 1].T
  wf0 = filter_W[:, :, 0].T
  wg1 = gate_W[:, :, 1].T
  wg0 = gate_W[:, :, 0].T
  bf = filter_b.reshape(1, C)
  bg = gate_b.reshape(1, C)
  bo = gcn_b.reshape(1, C)

  xws1 = _tc_a(xTp, d0, d1, wf1, wf0, wg1, wg0, bf, bg, gcn_W, 0, True)
  xws2 = _tc_a(xTp, d0, d1, wf1, wf0, wg1, wg0, bf, bg, gcn_W, TCH, False)
  agg1 = _sc_main(xws1, src_pad, dst_pad)
  agg2 = _sc_main(xws2, src_pad, dst_pad)
  y1 = _tc_c(agg1, xTp, d0, d1, bo, 0)
  y2 = _tc_c(agg2, xTp, d0, d1, bo, TCH)
  o1 = jnp.transpose(y1[:, :N, :], (1, 2, 0))
  o2 = jnp.transpose(y2[:, :N, :], (1, 2, 0))
  return jnp.concatenate([o1, o2], axis=2)


# speedup vs baseline: 1.6451x; 1.6451x over previous
import functools

import jax
import jax.numpy as jnp
from jax import lax
from jax.experimental import pallas as pl
from jax.experimental.pallas import tpu as pltpu
from jax.experimental.pallas import tpu_sc as plsc

N = 10000
C = 128
T = 12
E = 320000

NUM_SC = 2
NUM_TILES = 16
B = 128
PER_TILE = -(-E // (NUM_TILES * B)) * B
NB = PER_TILE // B
EP = NUM_TILES * PER_TILE
NP = 10112
ROWS_W = NP // NUM_TILES
CH = C // NUM_SC
DEG_W = 16
NBUF = 8
TCH = 6

BN = 1264
NBLK = NP // BN


def _tc_a_body(xt_ref, xp_ref, d0_ref, d1_ref, wf1_ref, wf0_ref, wg1_ref,
               wg0_ref, bf_ref, bg_ref, gw_ref, out_ref, *, first):
  t = pl.program_id(0)
  xt = xt_ref[0]
  if first:
    fac = jnp.where(t == 0, 0.0, 1.0).astype(jnp.float32)
    xp = xp_ref[0] * fac
  else:
    xp = xp_ref[0]
  dot = functools.partial(jnp.dot, preferred_element_type=jnp.float32)
  fpre = dot(xt, wf1_ref[...]) + dot(xp, wf0_ref[...]) + bf_ref[...]
  gpre = dot(xt, wg1_ref[...]) + dot(xp, wg0_ref[...]) + bg_ref[...]
  h = jnp.tanh(fpre) * jax.nn.sigmoid(gpre)
  xw = dot(h, gw_ref[...])
  dinv = lax.rsqrt(d0_ref[...] + d1_ref[...] + 1.0)
  xws = (xw * dinv).astype(jnp.bfloat16)
  out_ref[0, 0] = xws[:, :CH]
  out_ref[1, 0] = xws[:, CH:]


def _tc_a(xT, d0, d1, wf1, wf0, wg1, wg0, bf, bg, gw, toff, first):
  dblk = pl.BlockSpec((BN, 1), lambda t, nb: (nb, 0))
  wblk = pl.BlockSpec((C, C), lambda t, nb: (0, 0))
  bblk = pl.BlockSpec((1, C), lambda t, nb: (0, 0))
  if first:
    pmap = lambda t, nb: (jnp.where(t == 0, 0, t - 1), nb, 0)
  else:
    pmap = lambda t, nb: (t + toff - 1, nb, 0)
  return pl.pallas_call(
      functools.partial(_tc_a_body, first=first),
      grid=(TCH, NBLK),
      in_specs=[
          pl.BlockSpec((1, BN, C), lambda t, nb: (t + toff, nb, 0)),
          pl.BlockSpec((1, BN, C), pmap),
          dblk, dblk, wblk, wblk, wblk, wblk, bblk, bblk, wblk,
      ],
      out_specs=pl.BlockSpec((NUM_SC, 1, BN, CH), lambda t, nb: (0, t, nb, 0)),
      out_shape=jax.ShapeDtypeStruct((NUM_SC, TCH, NP, CH), jnp.bfloat16),
  )(xT, xT, d0, d1, wf1, wf0, wg1, wg0, bf, bg, gw)


def _sc_deg_body(dst_hbm, out_hbm, dstv, ones_v, zero_v, agg_sp, sem):
  cid = lax.axis_index("c")
  sid = lax.axis_index("s")
  pltpu.sync_copy(dst_hbm.at[sid], dstv)

  def fill_ones(i, _):
    ones_v[i, :] = jnp.full((DEG_W,), 1.0, jnp.float32)
    return 0
  lax.fori_loop(0, B, fill_ones, 0)

  def fill_zero(i, _):
    zero_v[i, :] = jnp.zeros((DEG_W,), jnp.float32)
    return 0
  lax.fori_loop(0, ROWS_W, fill_zero, 0)
  pltpu.sync_copy(zero_v, agg_sp.at[pl.ds(sid * ROWS_W, ROWS_W)])
  plsc.subcore_barrier()

  nb0 = NB // 2 + 1
  lo = jnp.where(cid == 0, 0, nb0)
  hi = jnp.where(cid == 0, nb0, NB)

  def batch(j, _):
    pltpu.sync_copy(ones_v, agg_sp.at[dstv.at[j]], add=True)
    return 0
  lax.fori_loop(lo, hi, batch, 0)
  plsc.subcore_barrier()
  pltpu.sync_copy(agg_sp.at[pl.ds(sid * ROWS_W, ROWS_W)],
                  out_hbm.at[cid].at[pl.ds(sid * ROWS_W, ROWS_W)])


def _sc_deg(dst_pad):
  mesh = plsc.VectorSubcoreMesh(core_axis_name="c", subcore_axis_name="s")
  return pl.kernel(
      _sc_deg_body,
      compiler_params=pltpu.CompilerParams(use_tc_tiling_on_sc=False),
      out_type=jax.ShapeDtypeStruct((NUM_SC, NP, DEG_W), jnp.float32),
      mesh=mesh,
      scratch_types=[
          pltpu.VMEM((NB, B), jnp.int32),
          pltpu.VMEM((B, DEG_W), jnp.float32),
          pltpu.VMEM((ROWS_W, DEG_W), jnp.float32),
          pltpu.VMEM_SHARED((NP, DEG_W), jnp.float32),
          pltpu.SemaphoreType.DMA,
      ],
  )(dst_pad)


def _sc_main_body(xws_hbm, src_hbm, dst_hbm, agg_hbm, srcv, dstv, rows, agg_sp,
                  gsem, ssem):
  cid = lax.axis_index("c")
  sid = lax.axis_index("s")
  pltpu.sync_copy(src_hbm.at[sid], srcv)
  pltpu.sync_copy(dst_hbm.at[sid], dstv)

  def per_t(t, _):
    pltpu.sync_copy(xws_hbm.at[cid].at[t].at[pl.ds(sid * ROWS_W, ROWS_W)],
                    agg_sp.at[pl.ds(sid * ROWS_W, ROWS_W)])
    plsc.subcore_barrier()

    tbl = xws_hbm.at[cid].at[t]
    for jj in range(NBUF - 1):
      pltpu.async_copy(tbl.at[srcv.at[jj]], rows.at[jj], gsem.at[jj])

    def batch(j, _):
      p = j % NBUF
      pr = (j + NBUF - 1) % NBUF

      @pl.when(j > 0)
      def _():
        pltpu.make_async_copy(rows.at[pr], agg_sp.at[dstv.at[j - 1]],
                              ssem.at[pr]).wait()

      @pl.when(j + NBUF - 1 < NB)
      def _():
        pltpu.async_copy(tbl.at[srcv.at[j + NBUF - 1]], rows.at[pr],
                         gsem.at[pr])

      pltpu.make_async_copy(tbl.at[srcv.at[j]], rows.at[p], gsem.at[p]).wait()
      pltpu.async_copy(rows.at[p], agg_sp.at[dstv.at[j]], ssem.at[p], add=True)
      return 0
    lax.fori_loop(0, NB, batch, 0)
    pltpu.make_async_copy(rows.at[(NB - 1) % NBUF], agg_sp.at[dstv.at[NB - 1]],
                          ssem.at[(NB - 1) % NBUF]).wait()
    plsc.subcore_barrier()
    pltpu.sync_copy(agg_sp.at[pl.ds(sid * ROWS_W, ROWS_W)],
                    agg_hbm.at[cid].at[t].at[pl.ds(sid * ROWS_W, ROWS_W)])
    return 0

  lax.fori_loop(0, TCH, per_t, 0)


def _sc_main(xws, src_pad, dst_pad):
  mesh = plsc.VectorSubcoreMesh(core_axis_name="c", subcore_axis_name="s")
  return pl.kernel(
      _sc_main_body,
      compiler_params=pltpu.CompilerParams(use_tc_tiling_on_sc=False),
      out_type=jax.ShapeDtypeStruct((NUM_SC, TCH, NP, CH), jnp.bfloat16),
      mesh=mesh,
      scratch_types=[
          pltpu.VMEM((NB, B), jnp.int32),
          pltpu.VMEM((NB, B), jnp.int32),
          pltpu.VMEM((NBUF, B, CH), jnp.bfloat16),
          pltpu.VMEM_SHARED((NP, CH), jnp.bfloat16),
          pltpu.SemaphoreType.DMA((NBUF,)),
          pltpu.SemaphoreType.DMA((NBUF,)),
      ],
  )(xws, src_pad, dst_pad)


def _tc_c_body(agg_ref, xt_ref, d0_ref, d1_ref, b_ref, out_ref):
  dinv = lax.rsqrt(d0_ref[...] + d1_ref[...] + 1.0)
  agg = jnp.concatenate([agg_ref[0, 0], agg_ref[1, 0]], axis=-1).astype(jnp.float32)
  out_ref[0] = agg * dinv + b_ref[...] + xt_ref[0]


def _tc_c(agg, xT, d0, d1, b, toff):
  dblk = pl.BlockSpec((BN, 1), lambda t, nb: (nb, 0))
  bblk = pl.BlockSpec((1, C), lambda t, nb: (0, 0))
  return pl.pallas_call(
      _tc_c_body,
      grid=(TCH, NBLK),
      in_specs=[pl.BlockSpec((NUM_SC, 1, BN, CH), lambda t, nb: (0, t, nb, 0)),
                pl.BlockSpec((1, BN, C), lambda t, nb: (t + toff, nb, 0)),
                dblk, dblk, bblk],
      out_specs=pl.BlockSpec((1, BN, C), lambda t, nb: (t, nb, 0)),
      out_shape=jax.ShapeDtypeStruct((TCH, NP, C), jnp.float32),
  )(agg, xT, d0, d1, b)


@jax.jit
def kernel(x, edge_index, filter_W, filter_b, gate_W, gate_b, gcn_W, gcn_b):
  xT = jnp.transpose(x, (2, 0, 1))
  xTp = jnp.pad(xT, ((0, 0), (0, NP - N), (0, 0)))

  src = edge_index[0]
  dst = edge_index[1]
  pad = EP - E
  src_pad = jnp.concatenate([src, jnp.zeros((pad,), jnp.int32)])
  dst_pad = jnp.concatenate([dst, jnp.full((pad,), N, jnp.int32)])
  src_pad = src_pad.reshape(NUM_TILES, NB, B)
  dst_pad = dst_pad.reshape(NUM_TILES, NB, B)

  deg_parts = _sc_deg(dst_pad)
  d0 = deg_parts[0, :, 0:1]
  d1 = deg_parts[1, :, 0:1]

  wf1 = filter_W[:, :, 1].T
  wf0 = filter_W[:, :, 0].T
  wg1 = gate_W[:, :, 1].T
  wg0 = gate_W[:, :, 0].T
  bf = filter_b.reshape(1, C)
  bg = gate_b.reshape(1, C)
  bo = gcn_b.reshape(1, C)

  xws1 = _tc_a(xTp, d0, d1, wf1, wf0, wg1, wg0, bf, bg, gcn_W, 0, True)
  xws2 = _tc_a(xTp, d0, d1, wf1, wf0, wg1, wg0, bf, bg, gcn_W, TCH, False)
  agg1 = _sc_main(xws1, src_pad, dst_pad)
  agg2 = _sc_main(xws2, src_pad, dst_pad)
  y1 = _tc_c(agg1, xTp, d0, d1, bo, 0)
  y2 = _tc_c(agg2, xTp, d0, d1, bo, TCH)
  o1 = jnp.transpose(y1[:, :N, :], (1, 2, 0))
  o2 = jnp.transpose(y2[:, :N, :], (1, 2, 0))
  return jnp.concatenate([o1, o2], axis=2)


# speedup vs baseline: 1.7924x; 1.0896x over previous
import functools

import jax
import jax.numpy as jnp
from jax import lax
from jax.experimental import pallas as pl
from jax.experimental.pallas import tpu as pltpu
from jax.experimental.pallas import tpu_sc as plsc

N = 10000
C = 128
T = 12
E = 320000

NUM_SC = 2
NUM_TILES = 16
B = 128
PER_TILE = -(-E // (NUM_TILES * B)) * B
NB = PER_TILE // B
EP = NUM_TILES * PER_TILE
NP = 10112
ROWS_W = NP // NUM_TILES
CH = C // NUM_SC
DEG_W = 16
NBUF = 8
TCH = 6

BN = 2000
NBLK = N // BN


def _tc_a_body(x_ref, d0_ref, d1_ref, wf1_ref, wf0_ref, wg1_ref,
               wg0_ref, bf_ref, bg_ref, gw_ref, out_ref):
  dot = functools.partial(jnp.dot, preferred_element_type=jnp.float32)
  dinv = lax.rsqrt(d0_ref[...] + d1_ref[...] + 1.0)
  wf1 = wf1_ref[...]
  wf0 = wf0_ref[...]
  wg1 = wg1_ref[...]
  wg0 = wg0_ref[...]
  gw = gw_ref[...]
  for t in range(T):
    xt = x_ref[t].astype(jnp.bfloat16)
    fpre = dot(xt, wf1) + bf_ref[...]
    gpre = dot(xt, wg1) + bg_ref[...]
    if t > 0:
      xp = x_ref[t - 1].astype(jnp.bfloat16)
      fpre = fpre + dot(xp, wf0)
      gpre = gpre + dot(xp, wg0)
    h = (jnp.tanh(fpre) * jax.nn.sigmoid(gpre)).astype(jnp.bfloat16)
    xw = dot(h, gw)
    xws = (xw * dinv).astype(jnp.bfloat16)
    out_ref[0, t] = xws[:, :CH]
    out_ref[1, t] = xws[:, CH:]


def _tc_a(xT, d0, d1, wf1, wf0, wg1, wg0, bf, bg, gw):
  dblk = pl.BlockSpec((BN, 1), lambda nb: (nb, 0))
  wblk = pl.BlockSpec((C, C), lambda nb: (0, 0))
  bblk = pl.BlockSpec((1, C), lambda nb: (0, 0))
  return pl.pallas_call(
      _tc_a_body,
      grid=(NBLK,),
      in_specs=[
          pl.BlockSpec((T, BN, C), lambda nb: (0, nb, 0)),
          dblk, dblk, wblk, wblk, wblk, wblk, bblk, bblk, wblk,
      ],
      out_specs=pl.BlockSpec((NUM_SC, T, BN, CH), lambda nb: (0, 0, nb, 0)),
      out_shape=jax.ShapeDtypeStruct((NUM_SC, T, NP, CH), jnp.bfloat16),
  )(xT, d0, d1, wf1, wf0, wg1, wg0, bf, bg, gw)


def _sc_deg_body(dst_hbm, out_hbm, dstv, ones_v, zero_v, agg_sp, sem):
  cid = lax.axis_index("c")
  sid = lax.axis_index("s")
  pltpu.sync_copy(dst_hbm.at[sid], dstv)

  def fill_ones(i, _):
    ones_v[i, :] = jnp.full((DEG_W,), 1.0, jnp.float32)
    return 0
  lax.fori_loop(0, B, fill_ones, 0)

  def fill_zero(i, _):
    zero_v[i, :] = jnp.zeros((DEG_W,), jnp.float32)
    return 0
  lax.fori_loop(0, ROWS_W, fill_zero, 0)
  pltpu.sync_copy(zero_v, agg_sp.at[pl.ds(sid * ROWS_W, ROWS_W)])
  plsc.subcore_barrier()

  nb0 = NB // 2 + 1
  lo = jnp.where(cid == 0, 0, nb0)
  hi = jnp.where(cid == 0, nb0, NB)

  def batch(j, _):
    pltpu.sync_copy(ones_v, agg_sp.at[dstv.at[j]], add=True)
    return 0
  lax.fori_loop(lo, hi, batch, 0)
  plsc.subcore_barrier()
  pltpu.sync_copy(agg_sp.at[pl.ds(sid * ROWS_W, ROWS_W)],
                  out_hbm.at[cid].at[pl.ds(sid * ROWS_W, ROWS_W)])


def _sc_deg(dst_pad):
  mesh = plsc.VectorSubcoreMesh(core_axis_name="c", subcore_axis_name="s")
  return pl.kernel(
      _sc_deg_body,
      compiler_params=pltpu.CompilerParams(use_tc_tiling_on_sc=False),
      out_type=jax.ShapeDtypeStruct((NUM_SC, NP, DEG_W), jnp.float32),
      mesh=mesh,
      scratch_types=[
          pltpu.VMEM((NB, B), jnp.int32),
          pltpu.VMEM((B, DEG_W), jnp.float32),
          pltpu.VMEM((ROWS_W, DEG_W), jnp.float32),
          pltpu.VMEM_SHARED((NP, DEG_W), jnp.float32),
          pltpu.SemaphoreType.DMA,
      ],
  )(dst_pad)


def _sc_main_body(xws_hbm, src_hbm, dst_hbm, agg_hbm, srcv, dstv, rows, agg_sp,
                  gsem, ssem, *, toff):
  cid = lax.axis_index("c")
  sid = lax.axis_index("s")
  pltpu.sync_copy(src_hbm.at[sid], srcv)
  pltpu.sync_copy(dst_hbm.at[sid], dstv)

  def per_t(t, _):
    tbl = xws_hbm.at[cid].at[t + toff]
    pltpu.sync_copy(tbl.at[pl.ds(sid * ROWS_W, ROWS_W)],
                    agg_sp.at[pl.ds(sid * ROWS_W, ROWS_W)])
    plsc.subcore_barrier()
    for jj in range(NBUF - 1):
      pltpu.async_copy(tbl.at[srcv.at[jj]], rows.at[jj], gsem.at[jj])

    def batch(j, _):
      p = j % NBUF
      pr = (j + NBUF - 1) % NBUF

      @pl.when(j > 0)
      def _():
        pltpu.make_async_copy(rows.at[pr], agg_sp.at[dstv.at[j - 1]],
                              ssem.at[pr]).wait()

      @pl.when(j + NBUF - 1 < NB)
      def _():
        pltpu.async_copy(tbl.at[srcv.at[j + NBUF - 1]], rows.at[pr],
                         gsem.at[pr])

      pltpu.make_async_copy(tbl.at[srcv.at[j]], rows.at[p], gsem.at[p]).wait()
      pltpu.async_copy(rows.at[p], agg_sp.at[dstv.at[j]], ssem.at[p], add=True)
      return 0
    lax.fori_loop(0, NB, batch, 0)
    pltpu.make_async_copy(rows.at[(NB - 1) % NBUF], agg_sp.at[dstv.at[NB - 1]],
                          ssem.at[(NB - 1) % NBUF]).wait()
    plsc.subcore_barrier()
    pltpu.sync_copy(agg_sp.at[pl.ds(sid * ROWS_W, ROWS_W)],
                    agg_hbm.at[cid].at[t].at[pl.ds(sid * ROWS_W, ROWS_W)])
    return 0

  lax.fori_loop(0, TCH, per_t, 0)


def _sc_main(xws, src_pad, dst_pad, toff):
  mesh = plsc.VectorSubcoreMesh(core_axis_name="c", subcore_axis_name="s")
  return pl.kernel(
      functools.partial(_sc_main_body, toff=toff),
      compiler_params=pltpu.CompilerParams(use_tc_tiling_on_sc=False),
      out_type=jax.ShapeDtypeStruct((NUM_SC, TCH, NP, CH), jnp.bfloat16),
      mesh=mesh,
      scratch_types=[
          pltpu.VMEM((NB, B), jnp.int32),
          pltpu.VMEM((NB, B), jnp.int32),
          pltpu.VMEM((NBUF, B, CH), jnp.bfloat16),
          pltpu.VMEM_SHARED((NP, CH), jnp.bfloat16),
          pltpu.SemaphoreType.DMA((NBUF,)),
          pltpu.SemaphoreType.DMA((NBUF,)),
      ],
  )(xws, src_pad, dst_pad)


def _tc_c_body(agg_ref, xt_ref, d0_ref, d1_ref, b_ref, out_ref):
  dinv = lax.rsqrt(d0_ref[...] + d1_ref[...] + 1.0)
  agg = jnp.concatenate([agg_ref[0, 0], agg_ref[1, 0]], axis=-1).astype(jnp.float32)
  out_ref[0] = agg * dinv + b_ref[...] + xt_ref[0]


def _tc_c(agg, xT, d0, d1, b, toff):
  dblk = pl.BlockSpec((BN, 1), lambda t, nb: (nb, 0))
  bblk = pl.BlockSpec((1, C), lambda t, nb: (0, 0))
  return pl.pallas_call(
      _tc_c_body,
      grid=(TCH, NBLK),
      in_specs=[pl.BlockSpec((NUM_SC, 1, BN, CH), lambda t, nb: (0, t, nb, 0)),
                pl.BlockSpec((1, BN, C), lambda t, nb: (t + toff, nb, 0)),
                dblk, dblk, bblk],
      out_specs=pl.BlockSpec((1, BN, C), lambda t, nb: (t, nb, 0)),
      out_shape=jax.ShapeDtypeStruct((TCH, N, C), jnp.float32),
  )(agg, xT, d0, d1, b)


@jax.jit
def kernel(x, edge_index, filter_W, filter_b, gate_W, gate_b, gcn_W, gcn_b):
  xT = jnp.transpose(x, (2, 0, 1))

  src = edge_index[0]
  dst = edge_index[1]
  pad = EP - E
  src_pad = jnp.concatenate([src, jnp.zeros((pad,), jnp.int32)])
  dst_pad = jnp.concatenate([dst, jnp.full((pad,), N, jnp.int32)])
  src_pad = src_pad.reshape(NUM_TILES, NB, B)
  dst_pad = dst_pad.reshape(NUM_TILES, NB, B)

  deg_parts = _sc_deg(dst_pad)
  d0 = deg_parts[0, :, 0:1]
  d1 = deg_parts[1, :, 0:1]

  wf1 = filter_W[:, :, 1].T.astype(jnp.bfloat16)
  wf0 = filter_W[:, :, 0].T.astype(jnp.bfloat16)
  wg1 = gate_W[:, :, 1].T.astype(jnp.bfloat16)
  wg0 = gate_W[:, :, 0].T.astype(jnp.bfloat16)
  gwb = gcn_W.astype(jnp.bfloat16)
  bf = filter_b.reshape(1, C)
  bg = gate_b.reshape(1, C)
  bo = gcn_b.reshape(1, C)

  xws = _tc_a(xT, d0, d1, wf1, wf0, wg1, wg0, bf, bg, gwb)
  agg1 = _sc_main(xws, src_pad, dst_pad, 0)
  agg2 = _sc_main(xws, src_pad, dst_pad, TCH)
  y1 = _tc_c(agg1, xT, d0, d1, bo, 0)
  y2 = _tc_c(agg2, xT, d0, d1, bo, TCH)
  o1 = jnp.transpose(y1, (1, 2, 0))
  o2 = jnp.transpose(y2, (1, 2, 0))
  return jnp.concatenate([o1, o2], axis=2)


# speedup vs baseline: 1.8118x; 1.0108x over previous
import functools

import jax
import jax.numpy as jnp
from jax import lax
from jax.experimental import pallas as pl
from jax.experimental.pallas import tpu as pltpu
from jax.experimental.pallas import tpu_sc as plsc

N = 10000
C = 128
T = 12
E = 320000

NUM_SC = 2
NUM_TILES = 16
B = 128
PER_TILE = -(-E // (NUM_TILES * B)) * B
NB = PER_TILE // B
EP = NUM_TILES * PER_TILE
NP = 10112
ROWS_W = NP // NUM_TILES
CH = C // NUM_SC
DEG_W = 16
NBUF = 8
TCH = 6

BN = 2000
NBLK = N // BN


def _tc_a_body(x_ref, d0_ref, d1_ref, wf1_ref, wf0_ref, wg1_ref,
               wg0_ref, bf_ref, bg_ref, gw_ref, out_ref):
  dot = functools.partial(jnp.dot, preferred_element_type=jnp.float32)
  dinv = lax.rsqrt(d0_ref[...] + d1_ref[...] + 1.0)
  wf1 = wf1_ref[...]
  wf0 = wf0_ref[...]
  wg1 = wg1_ref[...]
  wg0 = wg0_ref[...]
  gw = gw_ref[...]
  for t in range(T):
    xt = x_ref[t].astype(jnp.bfloat16)
    fpre = dot(xt, wf1) + bf_ref[...]
    gpre = dot(xt, wg1) + bg_ref[...]
    if t > 0:
      xp = x_ref[t - 1].astype(jnp.bfloat16)
      fpre = fpre + dot(xp, wf0)
      gpre = gpre + dot(xp, wg0)
    h = (jnp.tanh(fpre) * jax.nn.sigmoid(gpre)).astype(jnp.bfloat16)
    xw = dot(h, gw)
    xws = (xw * dinv).astype(jnp.bfloat16)
    out_ref[0, t] = xws[:, :CH]
    out_ref[1, t] = xws[:, CH:]


def _tc_a(xT, d0, d1, wf1, wf0, wg1, wg0, bf, bg, gw):
  dblk = pl.BlockSpec((BN, 1), lambda nb: (nb, 0))
  wblk = pl.BlockSpec((C, C), lambda nb: (0, 0))
  bblk = pl.BlockSpec((1, C), lambda nb: (0, 0))
  return pl.pallas_call(
      _tc_a_body,
      grid=(NBLK,),
      in_specs=[
          pl.BlockSpec((T, BN, C), lambda nb: (0, nb, 0)),
          dblk, dblk, wblk, wblk, wblk, wblk, bblk, bblk, wblk,
      ],
      out_specs=pl.BlockSpec((NUM_SC, T, BN, CH), lambda nb: (0, 0, nb, 0)),
      out_shape=jax.ShapeDtypeStruct((NUM_SC, T, NP, CH), jnp.bfloat16),
  )(xT, d0, d1, wf1, wf0, wg1, wg0, bf, bg, gw)


def _sc_deg_body(dst_hbm, out_hbm, dstv, ones_v, zero_v, agg_sp, sem):
  cid = lax.axis_index("c")
  sid = lax.axis_index("s")
  pltpu.sync_copy(dst_hbm.at[sid], dstv)

  def fill_ones(i, _):
    ones_v[i, :] = jnp.full((DEG_W,), 1.0, jnp.float32)
    return 0
  lax.fori_loop(0, B, fill_ones, 0)

  def fill_zero(i, _):
    zero_v[i, :] = jnp.zeros((DEG_W,), jnp.float32)
    return 0
  lax.fori_loop(0, ROWS_W, fill_zero, 0)
  pltpu.sync_copy(zero_v, agg_sp.at[pl.ds(sid * ROWS_W, ROWS_W)])
  plsc.subcore_barrier()

  nb0 = NB // 2 + 1
  lo = jnp.where(cid == 0, 0, nb0)
  hi = jnp.where(cid == 0, nb0, NB)

  def batch(j, _):
    pltpu.sync_copy(ones_v, agg_sp.at[dstv.at[j]], add=True)
    return 0
  lax.fori_loop(lo, hi, batch, 0)
  plsc.subcore_barrier()
  pltpu.sync_copy(agg_sp.at[pl.ds(sid * ROWS_W, ROWS_W)],
                  out_hbm.at[cid].at[pl.ds(sid * ROWS_W, ROWS_W)])


def _sc_deg(dst_pad):
  mesh = plsc.VectorSubcoreMesh(core_axis_name="c", subcore_axis_name="s")
  return pl.kernel(
      _sc_deg_body,
      compiler_params=pltpu.CompilerParams(use_tc_tiling_on_sc=False),
      out_type=jax.ShapeDtypeStruct((NUM_SC, NP, DEG_W), jnp.float32),
      mesh=mesh,
      scratch_types=[
          pltpu.VMEM((NB, B), jnp.int32),
          pltpu.VMEM((B, DEG_W), jnp.float32),
          pltpu.VMEM((ROWS_W, DEG_W), jnp.float32),
          pltpu.VMEM_SHARED((NP, DEG_W), jnp.float32),
          pltpu.SemaphoreType.DMA,
      ],
  )(dst_pad)


def _sc_main_body(xws_hbm, src_hbm, dst_hbm, agg_hbm, srcv, dstv, rows, agg_sp,
                  gsem, ssem, *, toff):
  cid = lax.axis_index("c")
  sid = lax.axis_index("s")
  pltpu.sync_copy(src_hbm.at[sid], srcv)
  pltpu.sync_copy(dst_hbm.at[sid], dstv)

  def per_t(t, _):
    tbl = xws_hbm.at[cid].at[t + toff]
    pltpu.sync_copy(tbl.at[pl.ds(sid * ROWS_W, ROWS_W)],
                    agg_sp.at[pl.ds(sid * ROWS_W, ROWS_W)])
    plsc.subcore_barrier()
    for jj in range(NBUF - 1):
      pltpu.async_copy(tbl.at[srcv.at[jj]], rows.at[jj], gsem.at[jj])

    def batch(j, _):
      p = j % NBUF
      pr = (j + NBUF - 1) % NBUF

      @pl.when(j > 0)
      def _():
        pltpu.make_async_copy(rows.at[pr], agg_sp.at[dstv.at[j - 1]],
                              ssem.at[pr]).wait()

      @pl.when(j + NBUF - 1 < NB)
      def _():
        pltpu.async_copy(tbl.at[srcv.at[j + NBUF - 1]], rows.at[pr],
                         gsem.at[pr])

      pltpu.make_async_copy(tbl.at[srcv.at[j]], rows.at[p], gsem.at[p]).wait()
      pltpu.async_copy(rows.at[p], agg_sp.at[dstv.at[j]], ssem.at[p], add=True)
      return 0
    lax.fori_loop(0, NB, batch, 0)
    pltpu.make_async_copy(rows.at[(NB - 1) % NBUF], agg_sp.at[dstv.at[NB - 1]],
                          ssem.at[(NB - 1) % NBUF]).wait()
    plsc.subcore_barrier()
    pltpu.sync_copy(agg_sp.at[pl.ds(sid * ROWS_W, ROWS_W)],
                    agg_hbm.at[cid].at[t].at[pl.ds(sid * ROWS_W, ROWS_W)])
    return 0

  lax.fori_loop(0, T, per_t, 0)


def _sc_main(xws, src_pad, dst_pad, toff):
  mesh = plsc.VectorSubcoreMesh(core_axis_name="c", subcore_axis_name="s")
  return pl.kernel(
      functools.partial(_sc_main_body, toff=toff),
      compiler_params=pltpu.CompilerParams(use_tc_tiling_on_sc=False),
      out_type=jax.ShapeDtypeStruct((NUM_SC, T, NP, CH), jnp.bfloat16),
      mesh=mesh,
      scratch_types=[
          pltpu.VMEM((NB, B), jnp.int32),
          pltpu.VMEM((NB, B), jnp.int32),
          pltpu.VMEM((NBUF, B, CH), jnp.bfloat16),
          pltpu.VMEM_SHARED((NP, CH), jnp.bfloat16),
          pltpu.SemaphoreType.DMA((NBUF,)),
          pltpu.SemaphoreType.DMA((NBUF,)),
      ],
  )(xws, src_pad, dst_pad)


def _tc_c_body(agg_ref, xt_ref, d0_ref, d1_ref, b_ref, out_ref):
  dinv = lax.rsqrt(d0_ref[...] + d1_ref[...] + 1.0)
  agg = jnp.concatenate([agg_ref[0, 0], agg_ref[1, 0]], axis=-1).astype(jnp.float32)
  out_ref[0] = agg * dinv + b_ref[...] + xt_ref[0]


def _tc_c(agg, xT, d0, d1, b):
  dblk = pl.BlockSpec((BN, 1), lambda t, nb: (nb, 0))
  bblk = pl.BlockSpec((1, C), lambda t, nb: (0, 0))
  return pl.pallas_call(
      _tc_c_body,
      grid=(T, NBLK),
      in_specs=[pl.BlockSpec((NUM_SC, 1, BN, CH), lambda t, nb: (0, t, nb, 0)),
                pl.BlockSpec((1, BN, C), lambda t, nb: (t, nb, 0)),
                dblk, dblk, bblk],
      out_specs=pl.BlockSpec((1, BN, C), lambda t, nb: (t, nb, 0)),
      out_shape=jax.ShapeDtypeStruct((T, N, C), jnp.float32),
  )(agg, xT, d0, d1, b)


@jax.jit
def kernel(x, edge_index, filter_W, filter_b, gate_W, gate_b, gcn_W, gcn_b):
  xT = jnp.transpose(x, (2, 0, 1))

  src = edge_index[0]
  dst = edge_index[1]
  pad = EP - E
  src_pad = jnp.concatenate([src, jnp.zeros((pad,), jnp.int32)])
  dst_pad = jnp.concatenate([dst, jnp.full((pad,), N, jnp.int32)])
  src_pad = src_pad.reshape(NUM_TILES, NB, B)
  dst_pad = dst_pad.reshape(NUM_TILES, NB, B)

  deg_parts = _sc_deg(dst_pad)
  d0 = deg_parts[0, :, 0:1]
  d1 = deg_parts[1, :, 0:1]

  wf1 = filter_W[:, :, 1].T.astype(jnp.bfloat16)
  wf0 = filter_W[:, :, 0].T.astype(jnp.bfloat16)
  wg1 = gate_W[:, :, 1].T.astype(jnp.bfloat16)
  wg0 = gate_W[:, :, 0].T.astype(jnp.bfloat16)
  gwb = gcn_W.astype(jnp.bfloat16)
  bf = filter_b.reshape(1, C)
  bg = gate_b.reshape(1, C)
  bo = gcn_b.reshape(1, C)

  xws = _tc_a(xT, d0, d1, wf1, wf0, wg1, wg0, bf, bg, gwb)
  agg = _sc_main(xws, src_pad, dst_pad, 0)
  y = _tc_c(agg, xT, d0, d1, bo)
  return jnp.transpose(y, (1, 2, 0))


# speedup vs baseline: 1.8126x; 1.0004x over previous
import functools

import jax
import jax.numpy as jnp
from jax import lax
from jax.experimental import pallas as pl
from jax.experimental.pallas import tpu as pltpu
from jax.experimental.pallas import tpu_sc as plsc

N = 10000
C = 128
T = 12
E = 320000

NUM_SC = 2
NUM_TILES = 16
B = 128
PER_TILE = -(-E // (NUM_TILES * B)) * B
NB = PER_TILE // B
EP = NUM_TILES * PER_TILE
NP = 10112
ROWS_W = NP // NUM_TILES
CH = C // NUM_SC
DEG_W = 16
NBUF = 8

BN = 2000
NBLK = N // BN


def _tc_a_body(x_ref, d0_ref, d1_ref, wf1_ref, wf0_ref, wg1_ref,
               wg0_ref, bf_ref, bg_ref, gw_ref, out_ref):
  dot = functools.partial(jnp.dot, preferred_element_type=jnp.float32)
  dinv = lax.rsqrt(d0_ref[...] + d1_ref[...] + 1.0)
  wf1 = wf1_ref[...]
  wf0 = wf0_ref[...]
  wg1 = wg1_ref[...]
  wg0 = wg0_ref[...]
  gw = gw_ref[...]
  for t in range(T):
    xt = x_ref[t].astype(jnp.bfloat16)
    fpre = dot(xt, wf1) + bf_ref[...]
    gpre = dot(xt, wg1) + bg_ref[...]
    if t > 0:
      xp = x_ref[t - 1].astype(jnp.bfloat16)
      fpre = fpre + dot(xp, wf0)
      gpre = gpre + dot(xp, wg0)
    h = (jnp.tanh(fpre) * jax.nn.sigmoid(gpre)).astype(jnp.bfloat16)
    xw = dot(h, gw)
    xws = (xw * dinv).astype(jnp.bfloat16)
    out_ref[0, t] = xws[:, :CH]
    out_ref[1, t] = xws[:, CH:]


def _tc_a(xT, d0, d1, wf1, wf0, wg1, wg0, bf, bg, gw):
  dblk = pl.BlockSpec((BN, 1), lambda nb: (nb, 0))
  wblk = pl.BlockSpec((C, C), lambda nb: (0, 0))
  bblk = pl.BlockSpec((1, C), lambda nb: (0, 0))
  return pl.pallas_call(
      _tc_a_body,
      grid=(NBLK,),
      in_specs=[
          pl.BlockSpec((T, BN, C), lambda nb: (0, nb, 0)),
          dblk, dblk, wblk, wblk, wblk, wblk, bblk, bblk, wblk,
      ],
      out_specs=pl.BlockSpec((NUM_SC, T, BN, CH), lambda nb: (0, 0, nb, 0)),
      out_shape=jax.ShapeDtypeStruct((NUM_SC, T, NP, CH), jnp.bfloat16),
  )(xT, d0, d1, wf1, wf0, wg1, wg0, bf, bg, gw)


def _sc_deg_body(dst_hbm, out_hbm, dstv, ones_v, zero_v, agg_sp, sem):
  cid = lax.axis_index("c")
  sid = lax.axis_index("s")
  pltpu.sync_copy(dst_hbm.at[sid], dstv)

  def fill_ones(i, _):
    ones_v[i, :] = jnp.full((DEG_W,), 1.0, jnp.float32)
    return 0
  lax.fori_loop(0, B, fill_ones, 0)

  def fill_zero(i, _):
    zero_v[i, :] = jnp.zeros((DEG_W,), jnp.float32)
    return 0
  lax.fori_loop(0, ROWS_W, fill_zero, 0)
  pltpu.sync_copy(zero_v, agg_sp.at[pl.ds(sid * ROWS_W, ROWS_W)])
  plsc.subcore_barrier()

  nb0 = NB // 2 + 1
  lo = jnp.where(cid == 0, 0, nb0)
  hi = jnp.where(cid == 0, nb0, NB)

  def batch(j, _):
    pltpu.sync_copy(ones_v, agg_sp.at[dstv.at[j]], add=True)
    return 0
  lax.fori_loop(lo, hi, batch, 0)
  plsc.subcore_barrier()
  pltpu.sync_copy(agg_sp.at[pl.ds(sid * ROWS_W, ROWS_W)],
                  out_hbm.at[cid].at[pl.ds(sid * ROWS_W, ROWS_W)])


def _sc_deg(dst_pad):
  mesh = plsc.VectorSubcoreMesh(core_axis_name="c", subcore_axis_name="s")
  return pl.kernel(
      _sc_deg_body,
      compiler_params=pltpu.CompilerParams(use_tc_tiling_on_sc=False),
      out_type=jax.ShapeDtypeStruct((NUM_SC, NP, DEG_W), jnp.float32),
      mesh=mesh,
      scratch_types=[
          pltpu.VMEM((NB, B), jnp.int32),
          pltpu.VMEM((B, DEG_W), jnp.float32),
          pltpu.VMEM((ROWS_W, DEG_W), jnp.float32),
          pltpu.VMEM_SHARED((NP, DEG_W), jnp.float32),
          pltpu.SemaphoreType.DMA,
      ],
  )(dst_pad)


def _sc_main_body(xws_hbm, src_hbm, dst_hbm, agg_hbm, srcv, dstv, rows, agg_sp,
                  gsem, ssem, *, toff):
  cid = lax.axis_index("c")
  sid = lax.axis_index("s")
  pltpu.sync_copy(src_hbm.at[sid], srcv)
  pltpu.sync_copy(dst_hbm.at[sid], dstv)

  def per_t(t, _):
    tbl = xws_hbm.at[cid].at[t + toff]
    pltpu.sync_copy(tbl.at[pl.ds(sid * ROWS_W, ROWS_W)],
                    agg_sp.at[pl.ds(sid * ROWS_W, ROWS_W)])
    plsc.subcore_barrier()
    for jj in range(NBUF - 1):
      pltpu.async_copy(tbl.at[srcv.at[jj]], rows.at[jj], gsem.at[jj])

    def batch(j, _):
      p = j % NBUF
      pr = (j + NBUF - 1) % NBUF

      @pl.when(j > 0)
      def _():
        pltpu.make_async_copy(rows.at[pr], agg_sp.at[dstv.at[j - 1]],
                              ssem.at[pr]).wait()

      @pl.when(j + NBUF - 1 < NB)
      def _():
        pltpu.async_copy(tbl.at[srcv.at[j + NBUF - 1]], rows.at[pr],
                         gsem.at[pr])

      pltpu.make_async_copy(tbl.at[srcv.at[j]], rows.at[p], gsem.at[p]).wait()
      pltpu.async_copy(rows.at[p], agg_sp.at[dstv.at[j]], ssem.at[p], add=True)
      return 0
    lax.fori_loop(0, NB, batch, 0)
    pltpu.make_async_copy(rows.at[(NB - 1) % NBUF], agg_sp.at[dstv.at[NB - 1]],
                          ssem.at[(NB - 1) % NBUF]).wait()
    plsc.subcore_barrier()
    pltpu.sync_copy(agg_sp.at[pl.ds(sid * ROWS_W, ROWS_W)],
                    agg_hbm.at[cid].at[t].at[pl.ds(sid * ROWS_W, ROWS_W)])
    return 0

  lax.fori_loop(0, T, per_t, 0)


def _sc_main(xws, src_pad, dst_pad, toff):
  mesh = plsc.VectorSubcoreMesh(core_axis_name="c", subcore_axis_name="s")
  return pl.kernel(
      functools.partial(_sc_main_body, toff=toff),
      compiler_params=pltpu.CompilerParams(use_tc_tiling_on_sc=False),
      out_type=jax.ShapeDtypeStruct((NUM_SC, T, NP, CH), jnp.bfloat16),
      mesh=mesh,
      scratch_types=[
          pltpu.VMEM((NB, B), jnp.int32),
          pltpu.VMEM((NB, B), jnp.int32),
          pltpu.VMEM((NBUF, B, CH), jnp.bfloat16),
          pltpu.VMEM_SHARED((NP, CH), jnp.bfloat16),
          pltpu.SemaphoreType.DMA((NBUF,)),
          pltpu.SemaphoreType.DMA((NBUF,)),
      ],
  )(xws, src_pad, dst_pad)


def _tc_c_body(agg_ref, xt_ref, d0_ref, d1_ref, b_ref, out_ref):
  dinv = lax.rsqrt(d0_ref[...] + d1_ref[...] + 1.0)
  agg = jnp.concatenate([agg_ref[0, 0], agg_ref[1, 0]], axis=-1).astype(jnp.float32)
  out_ref[0] = agg * dinv + b_ref[...] + xt_ref[0]


def _tc_c(agg, xT, d0, d1, b):
  dblk = pl.BlockSpec((BN, 1), lambda t, nb: (nb, 0))
  bblk = pl.BlockSpec((1, C), lambda t, nb: (0, 0))
  return pl.pallas_call(
      _tc_c_body,
      grid=(T, NBLK),
      in_specs=[pl.BlockSpec((NUM_SC, 1, BN, CH), lambda t, nb: (0, t, nb, 0)),
                pl.BlockSpec((1, BN, C), lambda t, nb: (t, nb, 0)),
                dblk, dblk, bblk],
      out_specs=pl.BlockSpec((1, BN, C), lambda t, nb: (t, nb, 0)),
      out_shape=jax.ShapeDtypeStruct((T, N, C), jnp.float32),
  )(agg, xT, d0, d1, b)


@jax.jit
def kernel(x, edge_index, filter_W, filter_b, gate_W, gate_b, gcn_W, gcn_b):
  xT = jnp.transpose(x, (2, 0, 1))

  src = edge_index[0]
  dst = edge_index[1]
  pad = EP - E
  src_pad = jnp.concatenate([src, jnp.zeros((pad,), jnp.int32)])
  dst_pad = jnp.concatenate([dst, jnp.full((pad,), N, jnp.int32)])
  src_pad = src_pad.reshape(NUM_TILES, NB, B)
  dst_pad = dst_pad.reshape(NUM_TILES, NB, B)

  deg_parts = _sc_deg(dst_pad)
  d0 = deg_parts[0, :, 0:1]
  d1 = deg_parts[1, :, 0:1]

  wf1 = filter_W[:, :, 1].T.astype(jnp.bfloat16)
  wf0 = filter_W[:, :, 0].T.astype(jnp.bfloat16)
  wg1 = gate_W[:, :, 1].T.astype(jnp.bfloat16)
  wg0 = gate_W[:, :, 0].T.astype(jnp.bfloat16)
  gwb = gcn_W.astype(jnp.bfloat16)
  bf = filter_b.reshape(1, C)
  bg = gate_b.reshape(1, C)
  bo = gcn_b.reshape(1, C)

  xws = _tc_a(xT, d0, d1, wf1, wf0, wg1, wg0, bf, bg, gwb)
  agg = _sc_main(xws, src_pad, dst_pad, 0)
  y = _tc_c(agg, xT, d0, d1, bo)
  return jnp.transpose(y, (1, 2, 0))
